# Initial kernel scaffold; baseline (speedup 1.0000x reference)
import jax
import jax.numpy as jnp
from jax.experimental import pallas as pl


def kernel(x, params, edge_index, arch_list):
    raise NotImplementedError("write your pallas kernel here")


# speedup vs baseline: 2.6207x; 2.6207x over previous
import functools

import jax
import jax.numpy as jnp
from jax.experimental import pallas as pl

N = 10000
D = 256
E = 160000
I1 = 128
O2 = 64
A0 = 1999
A1 = 7996

NP = 10240
RB = 256
NB = NP // RB

F32 = jnp.float32


def _dot(a, b):
    return jnp.dot(a, b, preferred_element_type=F32)


def _dot_t(a, b):
    return jax.lax.dot_general(a, b, (((1,), (1,)), ((), ())),
                               preferred_element_type=F32)


def _proj1_body(x_ref, w_ref, b_ref, degs_ref, q_ref, k_ref, v_ref, hp_ref,
                dinv_ref):
    x = x_ref[...]
    y = _dot(x, w_ref[...]) + b_ref[...]
    deg = degs_ref[...] + 1.0
    dinv = jax.lax.rsqrt(deg)
    q_ref[...] = y[:, 0:I1]
    k_ref[...] = y[:, I1:2 * I1]
    v_ref[...] = y[:, 2 * I1:3 * I1]
    hp_ref[...] = dinv * y[:, 3 * I1:4 * I1]
    dinv_ref[...] = dinv


def _proj1(xp, wcat, bcat, degs):
    return pl.pallas_call(
        _proj1_body,
        grid=(NB,),
        in_specs=[
            pl.BlockSpec((RB, D), lambda i: (i, 0)),
            pl.BlockSpec((D, 4 * I1), lambda i: (0, 0)),
            pl.BlockSpec((1, 4 * I1), lambda i: (0, 0)),
            pl.BlockSpec((RB, 1), lambda i: (i, 0)),
        ],
        out_specs=[
            pl.BlockSpec((RB, I1), lambda i: (i, 0)),
            pl.BlockSpec((RB, I1), lambda i: (i, 0)),
            pl.BlockSpec((RB, I1), lambda i: (i, 0)),
            pl.BlockSpec((RB, I1), lambda i: (i, 0)),
            pl.BlockSpec((RB, 1), lambda i: (i, 0)),
        ],
        out_shape=[
            jax.ShapeDtypeStruct((NP, I1), F32),
            jax.ShapeDtypeStruct((NP, I1), F32),
            jax.ShapeDtypeStruct((NP, I1), F32),
            jax.ShapeDtypeStruct((NP, I1), F32),
            jax.ShapeDtypeStruct((NP, 1), F32),
        ],
    )(xp, wcat, bcat, degs)


def _attn_body(q_ref, k_ref, v_ref, o_ref):
    q = q_ref[...]
    s = _dot_t(q, k_ref[...])
    col = jax.lax.broadcasted_iota(jnp.int32, s.shape, 1)
    s = jnp.where(col < N, s, -3e38)
    m = jnp.max(s, axis=1, keepdims=True)
    p = jnp.exp(s - m)
    denom = jnp.sum(p, axis=1, keepdims=True)
    o = _dot(p, v_ref[...])
    o_ref[...] = o / denom


def _attn(q, k, v, dh):
    return pl.pallas_call(
        _attn_body,
        grid=(NB,),
        in_specs=[
            pl.BlockSpec((RB, dh), lambda i: (i, 0)),
            pl.BlockSpec((NP, dh), lambda i: (0, 0)),
            pl.BlockSpec((NP, dh), lambda i: (0, 0)),
        ],
        out_specs=pl.BlockSpec((RB, dh), lambda i: (i, 0)),
        out_shape=jax.ShapeDtypeStruct((NP, dh), F32),
    )(q, k, v)


def _sq1_body(s0_ref, s1_ref, hp_ref, dinv_ref, gb_ref, rel_ref, w1_ref,
              b1_ref, w2_ref, b2_ref, q_ref, k_ref, v_ref, hp2_ref):
    dinv = dinv_ref[...]
    xg = dinv * (s0_ref[...] + s1_ref[...] + hp_ref[...]) + gb_ref[...]
    a = jax.nn.relu(jnp.concatenate([xg, rel_ref[...]], axis=1))
    xsq = jax.nn.relu(_dot(a, w1_ref[...]) + b1_ref[...])
    y = _dot(xsq, w2_ref[...]) + b2_ref[...]
    q_ref[...] = y[:, 0:O2]
    k_ref[...] = y[:, O2:2 * O2]
    v_ref[...] = y[:, 2 * O2:3 * O2]
    hp2_ref[...] = dinv * y[:, 3 * O2:4 * O2]


def _sq1(s0, s1, hp, dinv, gb, rel, w1t, b1, w2t, b2):
    return pl.pallas_call(
        _sq1_body,
        grid=(NB,),
        in_specs=[
            pl.BlockSpec((RB, I1), lambda i: (i, 0)),
            pl.BlockSpec((RB, I1), lambda i: (i, 0)),
            pl.BlockSpec((RB, I1), lambda i: (i, 0)),
            pl.BlockSpec((RB, 1), lambda i: (i, 0)),
            pl.BlockSpec((1, I1), lambda i: (0, 0)),
            pl.BlockSpec((RB, I1), lambda i: (i, 0)),
            pl.BlockSpec((2 * I1, I1), lambda i: (0, 0)),
            pl.BlockSpec((1, I1), lambda i: (0, 0)),
            pl.BlockSpec((I1, 4 * O2), lambda i: (0, 0)),
            pl.BlockSpec((1, 4 * O2), lambda i: (0, 0)),
        ],
        out_specs=[
            pl.BlockSpec((RB, O2), lambda i: (i, 0)),
            pl.BlockSpec((RB, O2), lambda i: (i, 0)),
            pl.BlockSpec((RB, O2), lambda i: (i, 0)),
            pl.BlockSpec((RB, O2), lambda i: (i, 0)),
        ],
        out_shape=[
            jax.ShapeDtypeStruct((NP, O2), F32),
            jax.ShapeDtypeStruct((NP, O2), F32),
            jax.ShapeDtypeStruct((NP, O2), F32),
            jax.ShapeDtypeStruct((NP, O2), F32),
        ],
    )(s0, s1, hp, dinv, gb, rel, w1t, b1, w2t, b2)


def _sq2_body(s0_ref, s1_ref, hp_ref, dinv_ref, gb_ref, rel_ref, w1_ref,
              b1_ref, o_ref):
    dinv = dinv_ref[...]
    xg = dinv * (s0_ref[...] + s1_ref[...] + hp_ref[...]) + gb_ref[...]
    a = jax.nn.relu(jnp.concatenate([xg, rel_ref[...]], axis=1))
    o_ref[...] = jax.nn.relu(_dot(a, w1_ref[...]) + b1_ref[...])


def _sq2(s0, s1, hp, dinv, gb, rel, w1t, b1):
    return pl.pallas_call(
        _sq2_body,
        grid=(NB,),
        in_specs=[
            pl.BlockSpec((RB, O2), lambda i: (i, 0)),
            pl.BlockSpec((RB, O2), lambda i: (i, 0)),
            pl.BlockSpec((RB, O2), lambda i: (i, 0)),
            pl.BlockSpec((RB, 1), lambda i: (i, 0)),
            pl.BlockSpec((1, O2), lambda i: (0, 0)),
            pl.BlockSpec((RB, O2), lambda i: (i, 0)),
            pl.BlockSpec((2 * O2, O2), lambda i: (0, 0)),
            pl.BlockSpec((1, O2), lambda i: (0, 0)),
        ],
        out_specs=pl.BlockSpec((RB, O2), lambda i: (i, 0)),
        out_shape=jax.ShapeDtypeStruct((NP, O2), F32),
    )(s0, s1, hp, dinv, gb, rel, w1t, b1)


BR = 2048


def _tail_body(xs_ref, br_ref, wr_ref, cb_ref, tww_ref, twb_ref, midw_ref,
               midb_ref, botw_ref, botb_ref, f1w_ref, f1b_ref, f2w_ref,
               f2b_ref, m2_ref, tree_ref, mavg_ref, bwf_ref, otw_ref,
               omid_ref, obot_ref, feat_ref, ogcn_ref):
    xs = xs_ref[...]
    tw = xs[0:1, :]
    row = jax.lax.broadcasted_iota(jnp.int32, (NP, 1), 0)
    wmid = jnp.where((row >= 1) & (row <= A0), 1.0 / A0, 0.0)
    mid_avg = jnp.sum(wmid * xs, axis=0, keepdims=True)

    b4 = br_ref[...]
    m = _dot(b4, wr_ref[...]) + cb_ref[...]
    grow = jax.lax.broadcasted_iota(jnp.int32, (BR, 1), 0)
    gmask = grow < A0
    mmax = jnp.max(jnp.where(gmask, m, -3e38))
    m1 = m / mmax
    rs = jnp.sum(jnp.abs(m1), axis=1, keepdims=True)
    m2 = jnp.where(gmask, m1 / rs, 0.0)
    tree = (m2[:, 0:1] * b4[:, 0:O2]
            + m2[:, 1:2] * b4[:, O2:2 * O2]
            + m2[:, 2:3] * b4[:, 2 * O2:3 * O2]
            + m2[:, 3:4] * b4[:, 3 * O2:4 * O2])
    bwf = jnp.sum(tree, axis=0, keepdims=True) * (1.0 / A0)

    otw = _dot(tw, tww_ref[...]) + twb_ref[...]
    omid = _dot(mid_avg, midw_ref[...]) + midb_ref[...]
    obot = _dot(bwf, botw_ref[...]) + botb_ref[...]
    feat = _dot(jnp.concatenate([tw, mid_avg, bwf], axis=1),
                f1w_ref[...]) + f1b_ref[...]
    ogcn = _dot(jax.nn.relu(feat), f2w_ref[...]) + f2b_ref[...]

    m2_ref[...] = m2
    tree_ref[...] = tree
    mavg_ref[...] = mid_avg
    bwf_ref[...] = bwf
    otw_ref[...] = otw
    omid_ref[...] = omid
    obot_ref[...] = obot
    feat_ref[...] = feat
    ogcn_ref[...] = ogcn


def _tail(xs, br, wr, cb, p):
    full = lambda shape: pl.BlockSpec(shape, lambda: tuple(0 for _ in shape))
    ins = [
        (xs, (NP, O2)), (br, (BR, 4 * O2)), (wr, (4 * O2, 4)), (cb, (1, 4)),
        (p['tw_w'].T, (O2, 4)), (p['tw_b'].reshape(1, 4), (1, 4)),
        (p['mid_w'].T, (O2, 4)), (p['mid_b'].reshape(1, 4), (1, 4)),
        (p['bot_w'].T, (O2, 4)), (p['bot_b'].reshape(1, 4), (1, 4)),
        (p['fc1_w'].T, (3 * O2, O2)), (p['fc1_b'].reshape(1, O2), (1, O2)),
        (p['fc2_w'].T, (O2, 4)), (p['fc2_b'].reshape(1, 4), (1, 4)),
    ]
    outs = [
        ((BR, 4), F32), ((BR, O2), F32), ((1, O2), F32), ((1, O2), F32),
        ((1, 4), F32), ((1, 4), F32), ((1, 4), F32), ((1, O2), F32),
        ((1, 4), F32),
    ]
    return pl.pallas_call(
        _tail_body,
        in_specs=[full(s) for _, s in ins],
        out_specs=[full(s) for s, _ in outs],
        out_shape=[jax.ShapeDtypeStruct(s, d) for s, d in outs],
    )(*[a for a, _ in ins])


def _edge_scatter(hp, src, dst, dm):
    s = jnp.zeros((NP, dm), F32).at[dst].add(hp[src])
    return s, jnp.zeros((NP, dm), F32)


def _edge_deg(dst):
    return jnp.zeros((NP,), F32).at[dst].add(1.0).reshape(NP, 1)


def kernel(x, params, edge_index, arch_list):
    p = params
    shift = ((arch_list[0] - A0) + (arch_list[1] - A1)).astype(F32)
    xp = jnp.pad(x + shift, ((0, NP - N), (0, 0)))
    src = edge_index[0]
    dst = edge_index[1]

    degs = _edge_deg(dst)

    wcat1 = jnp.concatenate(
        [p['q1_w'].T, p['k1_w'].T, p['v1_w'].T, p['g1_w'].T], axis=1)
    bcat1 = jnp.concatenate(
        [p['q1_b'], p['k1_b'], p['v1_b'], jnp.zeros((I1,), F32)]
    ).reshape(1, 4 * I1)
    q1, k1, v1, hp1, dinv = _proj1(xp, wcat1, bcat1, degs)

    rel1 = _attn(q1, k1, v1, I1)
    s1a, s1b = _edge_scatter(hp1, src, dst, I1)

    wcat2 = jnp.concatenate(
        [p['q2_w'].T, p['k2_w'].T, p['v2_w'].T, p['g2_w'].T], axis=1)
    bcat2 = jnp.concatenate(
        [p['q2_b'], p['k2_b'], p['v2_b'], jnp.zeros((O2,), F32)]
    ).reshape(1, 4 * O2)
    q2, k2, v2, hp2 = _sq1(s1a, s1b, hp1, dinv, p['g1_b'].reshape(1, I1),
                           rel1, p['sq1_w'].T, p['sq1_b'].reshape(1, I1),
                           wcat2, bcat2)

    rel2 = _attn(q2, k2, v2, O2)
    s2a, s2b = _edge_scatter(hp2, src, dst, O2)

    xsq2 = _sq2(s2a, s2b, hp2, dinv, p['g2_b'].reshape(1, O2), rel2,
                p['sq2_w'].T, p['sq2_b'].reshape(1, O2))

    bottom = xsq2[1 + A0:1 + A0 + A1]
    br = jnp.pad(bottom.reshape(A0, 4 * O2), ((0, BR - A0), (0, 0)))
    wr = jnp.transpose(p['conv_w'], (2, 1, 0)).reshape(4 * O2, 4)
    cb = p['conv_b'].reshape(1, 4)

    (m2, tree, mid_avg, bwf, otw, omid, obot, feat, ogcn) = _tail(
        xsq2, br, wr, cb, p)

    tw = xsq2[0:1, :]
    mid = xsq2[1:1 + A0, :]
    bwv = m2.reshape(4 * BR, 1)[:A1]
    tree_bottom = tree[:A0]
    return (ogcn, otw, omid, obot, feat, tw, mid_avg, bwf, bwv, mid,
            tree_bottom)


# speedup vs baseline: 6.0036x; 2.2908x over previous
import functools

import jax
import jax.numpy as jnp
from jax import lax
from jax.experimental import pallas as pl
from jax.experimental.pallas import tpu as pltpu
from jax.experimental.pallas import tpu_sc as plsc

N = 10000
D = 256
E = 160000
I1 = 128
O2 = 64
A0 = 1999
A1 = 7996

NP = 10240
RB = 256
NB = NP // RB

F32 = jnp.float32


def _dot(a, b):
    return jnp.dot(a, b, preferred_element_type=F32)


def _dot_t(a, b):
    return jax.lax.dot_general(a, b, (((1,), (1,)), ((), ())),
                               preferred_element_type=F32)


def _proj1_body(x_ref, w_ref, b_ref, degs_ref, q_ref, k_ref, v_ref, hp_ref,
                dinv_ref):
    x = x_ref[...]
    y = _dot(x, w_ref[...]) + b_ref[...]
    deg = degs_ref[...] + 1.0
    dinv = jax.lax.rsqrt(deg)
    q_ref[...] = y[:, 0:I1]
    k_ref[...] = y[:, I1:2 * I1]
    v_ref[...] = y[:, 2 * I1:3 * I1]
    hp_ref[...] = dinv * y[:, 3 * I1:4 * I1]
    dinv_ref[...] = dinv


def _proj1(xp, wcat, bcat, degs):
    return pl.pallas_call(
        _proj1_body,
        grid=(NB,),
        in_specs=[
            pl.BlockSpec((RB, D), lambda i: (i, 0)),
            pl.BlockSpec((D, 4 * I1), lambda i: (0, 0)),
            pl.BlockSpec((1, 4 * I1), lambda i: (0, 0)),
            pl.BlockSpec((RB, 1), lambda i: (i, 0)),
        ],
        out_specs=[
            pl.BlockSpec((RB, I1), lambda i: (i, 0)),
            pl.BlockSpec((RB, I1), lambda i: (i, 0)),
            pl.BlockSpec((RB, I1), lambda i: (i, 0)),
            pl.BlockSpec((RB, I1), lambda i: (i, 0)),
            pl.BlockSpec((RB, 1), lambda i: (i, 0)),
        ],
        out_shape=[
            jax.ShapeDtypeStruct((NP, I1), F32),
            jax.ShapeDtypeStruct((NP, I1), F32),
            jax.ShapeDtypeStruct((NP, I1), F32),
            jax.ShapeDtypeStruct((NP, I1), F32),
            jax.ShapeDtypeStruct((NP, 1), F32),
        ],
    )(xp, wcat, bcat, degs)


def _attn_body(q_ref, k_ref, v_ref, o_ref):
    q = q_ref[...]
    s = _dot_t(q, k_ref[...])
    col = jax.lax.broadcasted_iota(jnp.int32, s.shape, 1)
    s = jnp.where(col < N, s, -3e38)
    m = jnp.max(s, axis=1, keepdims=True)
    p = jnp.exp(s - m)
    denom = jnp.sum(p, axis=1, keepdims=True)
    o = _dot(p, v_ref[...])
    o_ref[...] = o / denom


def _attn(q, k, v, dh):
    return pl.pallas_call(
        _attn_body,
        grid=(NB,),
        in_specs=[
            pl.BlockSpec((RB, dh), lambda i: (i, 0)),
            pl.BlockSpec((NP, dh), lambda i: (0, 0)),
            pl.BlockSpec((NP, dh), lambda i: (0, 0)),
        ],
        out_specs=pl.BlockSpec((RB, dh), lambda i: (i, 0)),
        out_shape=jax.ShapeDtypeStruct((NP, dh), F32),
    )(q, k, v)


def _sq1_body(s0_ref, s1_ref, hp_ref, dinv_ref, gb_ref, rel_ref, w1_ref,
              b1_ref, w2_ref, b2_ref, q_ref, k_ref, v_ref, hp2_ref):
    dinv = dinv_ref[...]
    xg = dinv * (s0_ref[...] + s1_ref[...] + hp_ref[...]) + gb_ref[...]
    a = jax.nn.relu(jnp.concatenate([xg, rel_ref[...]], axis=1))
    xsq = jax.nn.relu(_dot(a, w1_ref[...]) + b1_ref[...])
    y = _dot(xsq, w2_ref[...]) + b2_ref[...]
    q_ref[...] = y[:, 0:O2]
    k_ref[...] = y[:, O2:2 * O2]
    v_ref[...] = y[:, 2 * O2:3 * O2]
    hp2_ref[...] = jnp.concatenate(
        [dinv * y[:, 3 * O2:4 * O2], jnp.zeros((RB, O2), F32)], axis=1)


def _sq1(s0, s1, hp, dinv, gb, rel, w1t, b1, w2t, b2):
    return pl.pallas_call(
        _sq1_body,
        grid=(NB,),
        in_specs=[
            pl.BlockSpec((RB, I1), lambda i: (i, 0)),
            pl.BlockSpec((RB, I1), lambda i: (i, 0)),
            pl.BlockSpec((RB, I1), lambda i: (i, 0)),
            pl.BlockSpec((RB, 1), lambda i: (i, 0)),
            pl.BlockSpec((1, I1), lambda i: (0, 0)),
            pl.BlockSpec((RB, I1), lambda i: (i, 0)),
            pl.BlockSpec((2 * I1, I1), lambda i: (0, 0)),
            pl.BlockSpec((1, I1), lambda i: (0, 0)),
            pl.BlockSpec((I1, 4 * O2), lambda i: (0, 0)),
            pl.BlockSpec((1, 4 * O2), lambda i: (0, 0)),
        ],
        out_specs=[
            pl.BlockSpec((RB, O2), lambda i: (i, 0)),
            pl.BlockSpec((RB, O2), lambda i: (i, 0)),
            pl.BlockSpec((RB, O2), lambda i: (i, 0)),
            pl.BlockSpec((RB, I1), lambda i: (i, 0)),
        ],
        out_shape=[
            jax.ShapeDtypeStruct((NP, O2), F32),
            jax.ShapeDtypeStruct((NP, O2), F32),
            jax.ShapeDtypeStruct((NP, O2), F32),
            jax.ShapeDtypeStruct((NP, I1), F32),
        ],
    )(s0, s1, hp, dinv, gb, rel, w1t, b1, w2t, b2)


def _sq2_body(s0_ref, s1_ref, hp_ref, dinv_ref, gb_ref, rel_ref, w1_ref,
              b1_ref, o_ref):
    dinv = dinv_ref[...]
    acc = s0_ref[...] + s1_ref[...] + hp_ref[...]
    xg = dinv * acc[:, :O2] + gb_ref[...]
    a = jax.nn.relu(jnp.concatenate([xg, rel_ref[...]], axis=1))
    o_ref[...] = jax.nn.relu(_dot(a, w1_ref[...]) + b1_ref[...])


def _sq2(s0, s1, hp, dinv, gb, rel, w1t, b1):
    return pl.pallas_call(
        _sq2_body,
        grid=(NB,),
        in_specs=[
            pl.BlockSpec((RB, I1), lambda i: (i, 0)),
            pl.BlockSpec((RB, I1), lambda i: (i, 0)),
            pl.BlockSpec((RB, I1), lambda i: (i, 0)),
            pl.BlockSpec((RB, 1), lambda i: (i, 0)),
            pl.BlockSpec((1, O2), lambda i: (0, 0)),
            pl.BlockSpec((RB, O2), lambda i: (i, 0)),
            pl.BlockSpec((2 * O2, O2), lambda i: (0, 0)),
            pl.BlockSpec((1, O2), lambda i: (0, 0)),
        ],
        out_specs=pl.BlockSpec((RB, O2), lambda i: (i, 0)),
        out_shape=jax.ShapeDtypeStruct((NP, O2), F32),
    )(s0, s1, hp, dinv, gb, rel, w1t, b1)


BR = 2048


def _tail_body(xs_ref, br_ref, wr_ref, cb_ref, tww_ref, twb_ref, midw_ref,
               midb_ref, botw_ref, botb_ref, f1w_ref, f1b_ref, f2w_ref,
               f2b_ref, m2_ref, tree_ref, mavg_ref, bwf_ref, otw_ref,
               omid_ref, obot_ref, feat_ref, ogcn_ref):
    xs = xs_ref[...]
    tw = xs[0:1, :]
    row = jax.lax.broadcasted_iota(jnp.int32, (NP, 1), 0)
    wmid = jnp.where((row >= 1) & (row <= A0), 1.0 / A0, 0.0)
    mid_avg = jnp.sum(wmid * xs, axis=0, keepdims=True)

    b4 = br_ref[...]
    m = _dot(b4, wr_ref[...]) + cb_ref[...]
    grow = jax.lax.broadcasted_iota(jnp.int32, (BR, 1), 0)
    gmask = grow < A0
    mmax = jnp.max(jnp.where(gmask, m, -3e38))
    m1 = m / mmax
    rs = jnp.sum(jnp.abs(m1), axis=1, keepdims=True)
    m2 = jnp.where(gmask, m1 / rs, 0.0)
    tree = (m2[:, 0:1] * b4[:, 0:O2]
            + m2[:, 1:2] * b4[:, O2:2 * O2]
            + m2[:, 2:3] * b4[:, 2 * O2:3 * O2]
            + m2[:, 3:4] * b4[:, 3 * O2:4 * O2])
    bwf = jnp.sum(tree, axis=0, keepdims=True) * (1.0 / A0)

    otw = _dot(tw, tww_ref[...]) + twb_ref[...]
    omid = _dot(mid_avg, midw_ref[...]) + midb_ref[...]
    obot = _dot(bwf, botw_ref[...]) + botb_ref[...]
    feat = _dot(jnp.concatenate([tw, mid_avg, bwf], axis=1),
                f1w_ref[...]) + f1b_ref[...]
    ogcn = _dot(jax.nn.relu(feat), f2w_ref[...]) + f2b_ref[...]

    m2_ref[...] = m2
    tree_ref[...] = tree
    mavg_ref[...] = mid_avg
    bwf_ref[...] = bwf
    otw_ref[...] = otw
    omid_ref[...] = omid
    obot_ref[...] = obot
    feat_ref[...] = feat
    ogcn_ref[...] = ogcn


def _tail(xs, br, wr, cb, p):
    full = lambda shape: pl.BlockSpec(shape, lambda: tuple(0 for _ in shape))
    ins = [
        (xs, (NP, O2)), (br, (BR, 4 * O2)), (wr, (4 * O2, 4)), (cb, (1, 4)),
        (p['tw_w'].T, (O2, 4)), (p['tw_b'].reshape(1, 4), (1, 4)),
        (p['mid_w'].T, (O2, 4)), (p['mid_b'].reshape(1, 4), (1, 4)),
        (p['bot_w'].T, (O2, 4)), (p['bot_b'].reshape(1, 4), (1, 4)),
        (p['fc1_w'].T, (3 * O2, O2)), (p['fc1_b'].reshape(1, O2), (1, O2)),
        (p['fc2_w'].T, (O2, 4)), (p['fc2_b'].reshape(1, 4), (1, 4)),
    ]
    outs = [
        ((BR, 4), F32), ((BR, O2), F32), ((1, O2), F32), ((1, O2), F32),
        ((1, 4), F32), ((1, 4), F32), ((1, 4), F32), ((1, O2), F32),
        ((1, 4), F32),
    ]
    return pl.pallas_call(
        _tail_body,
        in_specs=[full(s) for _, s in ins],
        out_specs=[full(s) for s, _ in outs],
        out_shape=[jax.ShapeDtypeStruct(s, d) for s, d in outs],
    )(*[a for a, _ in ins])


NW = 32
CH = 128
NCH = 40
EP = NW * NCH * CH
RT = NP // 16
ZR = 32


def _zero_fill(z_v, s, acc, dm):
    zv = jnp.zeros((16,), F32)

    @pl.loop(0, ZR)
    def _(r):
        for jj in range(dm // 16):
            z_v[r, pl.ds(jj * 16, 16)] = zv

    @pl.loop(0, RT // ZR)
    def _(t):
        pltpu.sync_copy(z_v, acc.at[pl.ds(s * RT + t * ZR, ZR)])


def _sc_scatter(dm):
    mesh = plsc.VectorSubcoreMesh(core_axis_name="c", subcore_axis_name="s")

    def body(hp_hbm, srcg, dstg, out_hbm, src_v, dst_v, rows_v, z_v, acc, sem):
        c = lax.axis_index("c")
        s = lax.axis_index("s")
        wid = s * 2 + c
        _zero_fill(z_v, s, acc, dm)
        pltpu.sync_copy(srcg.at[wid], src_v)
        pltpu.sync_copy(dstg.at[wid], dst_v)
        plsc.subcore_barrier()

        @pl.loop(0, NCH)
        def _(j):
            pltpu.async_copy(hp_hbm.at[src_v.at[j]], rows_v, sem).wait()
            pltpu.sync_copy(rows_v, acc.at[dst_v.at[j]], add=True)

        plsc.subcore_barrier()
        pltpu.sync_copy(acc.at[pl.ds(s * RT, RT)],
                        out_hbm.at[c].at[pl.ds(s * RT, RT)])

    return pl.kernel(
        body,
        out_type=jax.ShapeDtypeStruct((2, NP, dm), F32),
        mesh=mesh,
        scratch_types=[
            pltpu.VMEM((NCH, CH), jnp.int32),
            pltpu.VMEM((NCH, CH), jnp.int32),
            pltpu.VMEM((CH, dm), F32),
            pltpu.VMEM((ZR, dm), F32),
            pltpu.VMEM_SHARED((NP, dm), F32),
            pltpu.SemaphoreType.DMA,
        ],
    )


def _sc_deg():
    mesh = plsc.VectorSubcoreMesh(core_axis_name="c", subcore_axis_name="s")

    def body(dstg, out_hbm, dst_v, ones_v, z_v, acc):
        c = lax.axis_index("c")
        s = lax.axis_index("s")
        wid = s * 2 + c
        _zero_fill(z_v, s, acc, I1)
        ov = jnp.ones((16,), F32)

        @pl.loop(0, CH)
        def _(r):
            for jj in range(I1 // 16):
                ones_v[r, pl.ds(jj * 16, 16)] = ov

        pltpu.sync_copy(dstg.at[wid], dst_v)
        plsc.subcore_barrier()

        @pl.loop(0, NCH)
        def _(j):
            pltpu.sync_copy(ones_v, acc.at[dst_v.at[j]], add=True)

        plsc.subcore_barrier()
        pltpu.sync_copy(acc.at[pl.ds(s * RT, RT)],
                        out_hbm.at[c].at[pl.ds(s * RT, RT)])

    return pl.kernel(
        body,
        out_type=jax.ShapeDtypeStruct((2, NP, I1), F32),
        mesh=mesh,
        scratch_types=[
            pltpu.VMEM((NCH, CH), jnp.int32),
            pltpu.VMEM((CH, I1), F32),
            pltpu.VMEM((ZR, I1), F32),
            pltpu.VMEM_SHARED((NP, I1), F32),
        ],
    )


def kernel(x, params, edge_index, arch_list):
    p = params
    shift = ((arch_list[0] - A0) + (arch_list[1] - A1)).astype(F32)
    xp = jnp.pad(x + shift, ((0, NP - N), (0, 0)))
    ei = jnp.concatenate(
        [edge_index,
         jnp.full((2, EP - E), NP - 1, edge_index.dtype)], axis=1)
    srcg = ei[0].reshape(NW, NCH, CH)
    dstg = ei[1].reshape(NW, NCH, CH)

    degp = _sc_deg()(dstg)
    degs = degp[0, :, :1] + degp[1, :, :1]

    wcat1 = jnp.concatenate(
        [p['q1_w'].T, p['k1_w'].T, p['v1_w'].T, p['g1_w'].T], axis=1)
    bcat1 = jnp.concatenate(
        [p['q1_b'], p['k1_b'], p['v1_b'], jnp.zeros((I1,), F32)]
    ).reshape(1, 4 * I1)
    q1, k1, v1, hp1, dinv = _proj1(xp, wcat1, bcat1, degs)

    rel1 = _attn(q1, k1, v1, I1)
    s1p = _sc_scatter(I1)(hp1, srcg, dstg)
    s1a, s1b = s1p[0], s1p[1]

    wcat2 = jnp.concatenate(
        [p['q2_w'].T, p['k2_w'].T, p['v2_w'].T, p['g2_w'].T], axis=1)
    bcat2 = jnp.concatenate(
        [p['q2_b'], p['k2_b'], p['v2_b'], jnp.zeros((O2,), F32)]
    ).reshape(1, 4 * O2)
    q2, k2, v2, hp2 = _sq1(s1a, s1b, hp1, dinv, p['g1_b'].reshape(1, I1),
                           rel1, p['sq1_w'].T, p['sq1_b'].reshape(1, I1),
                           wcat2, bcat2)

    rel2 = _attn(q2, k2, v2, O2)
    s2p = _sc_scatter(I1)(hp2, srcg, dstg)
    s2a, s2b = s2p[0], s2p[1]

    xsq2 = _sq2(s2a, s2b, hp2, dinv, p['g2_b'].reshape(1, O2), rel2,
                p['sq2_w'].T, p['sq2_b'].reshape(1, O2))

    bottom = xsq2[1 + A0:1 + A0 + A1]
    br = jnp.pad(bottom.reshape(A0, 4 * O2), ((0, BR - A0), (0, 0)))
    wr = jnp.transpose(p['conv_w'], (2, 1, 0)).reshape(4 * O2, 4)
    cb = p['conv_b'].reshape(1, 4)

    (m2, tree, mid_avg, bwf, otw, omid, obot, feat, ogcn) = _tail(
        xsq2, br, wr, cb, p)

    tw = xsq2[0:1, :]
    mid = xsq2[1:1 + A0, :]
    bwv = m2.reshape(4 * BR, 1)[:A1]
    tree_bottom = tree[:A0]
    return (ogcn, otw, omid, obot, feat, tw, mid_avg, bwf, bwv, mid,
            tree_bottom)


# speedup vs baseline: 6.2225x; 1.0365x over previous
import functools

import jax
import jax.numpy as jnp
from jax import lax
from jax.experimental import pallas as pl
from jax.experimental.pallas import tpu as pltpu
from jax.experimental.pallas import tpu_sc as plsc

N = 10000
D = 256
E = 160000
I1 = 128
O2 = 64
A0 = 1999
A1 = 7996

NP = 10240
RB = 256
NB = NP // RB

F32 = jnp.float32


def _dot(a, b):
    return jnp.dot(a, b, preferred_element_type=F32)


def _dot_t(a, b):
    return jax.lax.dot_general(a, b, (((1,), (1,)), ((), ())),
                               preferred_element_type=F32)


def _proj1_body(x_ref, w_ref, b_ref, degs_ref, q_ref, k_ref, v_ref, hp_ref,
                dinv_ref):
    x = x_ref[...]
    y = _dot(x, w_ref[...]) + b_ref[...]
    deg = degs_ref[...] + 1.0
    dinv = jax.lax.rsqrt(deg)
    q_ref[...] = y[:, 0:I1]
    k_ref[...] = y[:, I1:2 * I1]
    v_ref[...] = y[:, 2 * I1:3 * I1]
    hp_ref[...] = dinv * y[:, 3 * I1:4 * I1]
    dinv_ref[...] = dinv


def _proj1(xp, wcat, bcat, degs):
    return pl.pallas_call(
        _proj1_body,
        grid=(NB,),
        in_specs=[
            pl.BlockSpec((RB, D), lambda i: (i, 0)),
            pl.BlockSpec((D, 4 * I1), lambda i: (0, 0)),
            pl.BlockSpec((1, 4 * I1), lambda i: (0, 0)),
            pl.BlockSpec((RB, 1), lambda i: (i, 0)),
        ],
        out_specs=[
            pl.BlockSpec((RB, I1), lambda i: (i, 0)),
            pl.BlockSpec((RB, I1), lambda i: (i, 0)),
            pl.BlockSpec((RB, I1), lambda i: (i, 0)),
            pl.BlockSpec((RB, I1), lambda i: (i, 0)),
            pl.BlockSpec((RB, 1), lambda i: (i, 0)),
        ],
        out_shape=[
            jax.ShapeDtypeStruct((NP, I1), F32),
            jax.ShapeDtypeStruct((NP, I1), F32),
            jax.ShapeDtypeStruct((NP, I1), F32),
            jax.ShapeDtypeStruct((NP, I1), F32),
            jax.ShapeDtypeStruct((NP, 1), F32),
        ],
    )(xp, wcat, bcat, degs)


def _attn_body(q_ref, k_ref, v_ref, o_ref):
    q = q_ref[...]
    s = _dot_t(q, k_ref[...])
    col = jax.lax.broadcasted_iota(jnp.int32, s.shape, 1)
    s = jnp.where(col < N, s, -3e38)
    m = jnp.max(s, axis=1, keepdims=True)
    p = jnp.exp(s - m)
    denom = jnp.sum(p, axis=1, keepdims=True)
    o = _dot(p, v_ref[...])
    o_ref[...] = o / denom


def _attn(q, k, v, dh):
    return pl.pallas_call(
        _attn_body,
        grid=(NB,),
        in_specs=[
            pl.BlockSpec((RB, dh), lambda i: (i, 0)),
            pl.BlockSpec((NP, dh), lambda i: (0, 0)),
            pl.BlockSpec((NP, dh), lambda i: (0, 0)),
        ],
        out_specs=pl.BlockSpec((RB, dh), lambda i: (i, 0)),
        out_shape=jax.ShapeDtypeStruct((NP, dh), F32),
    )(q, k, v)


def _sq1_body(s0_ref, s1_ref, hp_ref, dinv_ref, gb_ref, rel_ref, w1_ref,
              b1_ref, w2_ref, b2_ref, q_ref, k_ref, v_ref, hp2_ref):
    dinv = dinv_ref[...]
    xg = dinv * (s0_ref[...] + s1_ref[...] + hp_ref[...]) + gb_ref[...]
    a = jax.nn.relu(jnp.concatenate([xg, rel_ref[...]], axis=1))
    xsq = jax.nn.relu(_dot(a, w1_ref[...]) + b1_ref[...])
    y = _dot(xsq, w2_ref[...]) + b2_ref[...]
    q_ref[...] = y[:, 0:O2]
    k_ref[...] = y[:, O2:2 * O2]
    v_ref[...] = y[:, 2 * O2:3 * O2]
    hp2_ref[...] = jnp.concatenate(
        [dinv * y[:, 3 * O2:4 * O2], jnp.zeros((RB, O2), F32)], axis=1)


def _sq1(s0, s1, hp, dinv, gb, rel, w1t, b1, w2t, b2):
    return pl.pallas_call(
        _sq1_body,
        grid=(NB,),
        in_specs=[
            pl.BlockSpec((RB, I1), lambda i: (i, 0)),
            pl.BlockSpec((RB, I1), lambda i: (i, 0)),
            pl.BlockSpec((RB, I1), lambda i: (i, 0)),
            pl.BlockSpec((RB, 1), lambda i: (i, 0)),
            pl.BlockSpec((1, I1), lambda i: (0, 0)),
            pl.BlockSpec((RB, I1), lambda i: (i, 0)),
            pl.BlockSpec((2 * I1, I1), lambda i: (0, 0)),
            pl.BlockSpec((1, I1), lambda i: (0, 0)),
            pl.BlockSpec((I1, 4 * O2), lambda i: (0, 0)),
            pl.BlockSpec((1, 4 * O2), lambda i: (0, 0)),
        ],
        out_specs=[
            pl.BlockSpec((RB, O2), lambda i: (i, 0)),
            pl.BlockSpec((RB, O2), lambda i: (i, 0)),
            pl.BlockSpec((RB, O2), lambda i: (i, 0)),
            pl.BlockSpec((RB, I1), lambda i: (i, 0)),
        ],
        out_shape=[
            jax.ShapeDtypeStruct((NP, O2), F32),
            jax.ShapeDtypeStruct((NP, O2), F32),
            jax.ShapeDtypeStruct((NP, O2), F32),
            jax.ShapeDtypeStruct((NP, I1), F32),
        ],
    )(s0, s1, hp, dinv, gb, rel, w1t, b1, w2t, b2)


def _sq2_body(s0_ref, s1_ref, hp_ref, dinv_ref, gb_ref, rel_ref, w1_ref,
              b1_ref, o_ref):
    dinv = dinv_ref[...]
    acc = s0_ref[...] + s1_ref[...] + hp_ref[...]
    xg = dinv * acc[:, :O2] + gb_ref[...]
    a = jax.nn.relu(jnp.concatenate([xg, rel_ref[...]], axis=1))
    o_ref[...] = jax.nn.relu(_dot(a, w1_ref[...]) + b1_ref[...])


def _sq2(s0, s1, hp, dinv, gb, rel, w1t, b1):
    return pl.pallas_call(
        _sq2_body,
        grid=(NB,),
        in_specs=[
            pl.BlockSpec((RB, I1), lambda i: (i, 0)),
            pl.BlockSpec((RB, I1), lambda i: (i, 0)),
            pl.BlockSpec((RB, I1), lambda i: (i, 0)),
            pl.BlockSpec((RB, 1), lambda i: (i, 0)),
            pl.BlockSpec((1, O2), lambda i: (0, 0)),
            pl.BlockSpec((RB, O2), lambda i: (i, 0)),
            pl.BlockSpec((2 * O2, O2), lambda i: (0, 0)),
            pl.BlockSpec((1, O2), lambda i: (0, 0)),
        ],
        out_specs=pl.BlockSpec((RB, O2), lambda i: (i, 0)),
        out_shape=jax.ShapeDtypeStruct((NP, O2), F32),
    )(s0, s1, hp, dinv, gb, rel, w1t, b1)


BR = 2048


def _tail_body(xs_ref, br_ref, wr_ref, cb_ref, tww_ref, twb_ref, midw_ref,
               midb_ref, botw_ref, botb_ref, f1w_ref, f1b_ref, f2w_ref,
               f2b_ref, m2_ref, tree_ref, mavg_ref, bwf_ref, otw_ref,
               omid_ref, obot_ref, feat_ref, ogcn_ref):
    xs = xs_ref[...]
    tw = xs[0:1, :]
    row = jax.lax.broadcasted_iota(jnp.int32, (NP, 1), 0)
    wmid = jnp.where((row >= 1) & (row <= A0), 1.0 / A0, 0.0)
    mid_avg = jnp.sum(wmid * xs, axis=0, keepdims=True)

    b4 = br_ref[...]
    m = _dot(b4, wr_ref[...]) + cb_ref[...]
    grow = jax.lax.broadcasted_iota(jnp.int32, (BR, 1), 0)
    gmask = grow < A0
    mmax = jnp.max(jnp.where(gmask, m, -3e38))
    m1 = m / mmax
    rs = jnp.sum(jnp.abs(m1), axis=1, keepdims=True)
    m2 = jnp.where(gmask, m1 / rs, 0.0)
    tree = (m2[:, 0:1] * b4[:, 0:O2]
            + m2[:, 1:2] * b4[:, O2:2 * O2]
            + m2[:, 2:3] * b4[:, 2 * O2:3 * O2]
            + m2[:, 3:4] * b4[:, 3 * O2:4 * O2])
    bwf = jnp.sum(tree, axis=0, keepdims=True) * (1.0 / A0)

    otw = _dot(tw, tww_ref[...]) + twb_ref[...]
    omid = _dot(mid_avg, midw_ref[...]) + midb_ref[...]
    obot = _dot(bwf, botw_ref[...]) + botb_ref[...]
    feat = _dot(jnp.concatenate([tw, mid_avg, bwf], axis=1),
                f1w_ref[...]) + f1b_ref[...]
    ogcn = _dot(jax.nn.relu(feat), f2w_ref[...]) + f2b_ref[...]

    m2_ref[...] = m2
    tree_ref[...] = tree
    mavg_ref[...] = mid_avg
    bwf_ref[...] = bwf
    otw_ref[...] = otw
    omid_ref[...] = omid
    obot_ref[...] = obot
    feat_ref[...] = feat
    ogcn_ref[...] = ogcn


def _tail(xs, br, wr, cb, p):
    full = lambda shape: pl.BlockSpec(shape, lambda: tuple(0 for _ in shape))
    ins = [
        (xs, (NP, O2)), (br, (BR, 4 * O2)), (wr, (4 * O2, 4)), (cb, (1, 4)),
        (p['tw_w'].T, (O2, 4)), (p['tw_b'].reshape(1, 4), (1, 4)),
        (p['mid_w'].T, (O2, 4)), (p['mid_b'].reshape(1, 4), (1, 4)),
        (p['bot_w'].T, (O2, 4)), (p['bot_b'].reshape(1, 4), (1, 4)),
        (p['fc1_w'].T, (3 * O2, O2)), (p['fc1_b'].reshape(1, O2), (1, O2)),
        (p['fc2_w'].T, (O2, 4)), (p['fc2_b'].reshape(1, 4), (1, 4)),
    ]
    outs = [
        ((BR, 4), F32), ((BR, O2), F32), ((1, O2), F32), ((1, O2), F32),
        ((1, 4), F32), ((1, 4), F32), ((1, 4), F32), ((1, O2), F32),
        ((1, 4), F32),
    ]
    return pl.pallas_call(
        _tail_body,
        in_specs=[full(s) for _, s in ins],
        out_specs=[full(s) for s, _ in outs],
        out_shape=[jax.ShapeDtypeStruct(s, d) for s, d in outs],
    )(*[a for a, _ in ins])


NW = 32
CH = 128
NCH = 40
EP = NW * NCH * CH
RT = NP // 16
ZR = 32


def _zero_fill(z_v, s, acc, dm):
    zv = jnp.zeros((16,), F32)

    @pl.loop(0, ZR)
    def _(r):
        for jj in range(dm // 16):
            z_v[r, pl.ds(jj * 16, 16)] = zv

    @pl.loop(0, RT // ZR)
    def _(t):
        pltpu.sync_copy(z_v, acc.at[pl.ds(s * RT + t * ZR, ZR)])


def _sc_scatter(dm):
    mesh = plsc.VectorSubcoreMesh(core_axis_name="c", subcore_axis_name="s")

    def body(hp_hbm, srcg, dstg, out_hbm, src_v, dst_v, rows0, rows1, z_v,
             acc, sem0, sem1):
        c = lax.axis_index("c")
        s = lax.axis_index("s")
        wid = s * 2 + c
        _zero_fill(z_v, s, acc, dm)
        pltpu.sync_copy(srcg.at[wid], src_v)
        pltpu.sync_copy(dstg.at[wid], dst_v)
        plsc.subcore_barrier()

        def gather(j, buf, sem):
            pltpu.async_copy(hp_hbm.at[src_v.at[j]], buf, sem)

        def drain(buf, sem):
            pltpu.make_async_copy(hp_hbm.at[src_v.at[0]], buf, sem).wait()

        def scat(j, buf):
            pltpu.sync_copy(buf, acc.at[dst_v.at[j]], add=True)

        gather(0, rows0, sem0)

        @pl.loop(0, NCH, step=2)
        def _(j):
            gather(j + 1, rows1, sem1)
            drain(rows0, sem0)
            scat(j, rows0)

            @pl.when(j + 2 < NCH)
            def _():
                gather(j + 2, rows0, sem0)

            drain(rows1, sem1)
            scat(j + 1, rows1)

        plsc.subcore_barrier()
        pltpu.sync_copy(acc.at[pl.ds(s * RT, RT)],
                        out_hbm.at[c].at[pl.ds(s * RT, RT)])

    return pl.kernel(
        body,
        out_type=jax.ShapeDtypeStruct((2, NP, dm), F32),
        mesh=mesh,
        scratch_types=[
            pltpu.VMEM((NCH, CH), jnp.int32),
            pltpu.VMEM((NCH, CH), jnp.int32),
            pltpu.VMEM((CH, dm), F32),
            pltpu.VMEM((CH, dm), F32),
            pltpu.VMEM((ZR, dm), F32),
            pltpu.VMEM_SHARED((NP, dm), F32),
            pltpu.SemaphoreType.DMA,
            pltpu.SemaphoreType.DMA,
        ],
    )


def _sc_deg():
    mesh = plsc.VectorSubcoreMesh(core_axis_name="c", subcore_axis_name="s")

    def body(dstg, out_hbm, dst_v, ones_v, z_v, acc):
        c = lax.axis_index("c")
        s = lax.axis_index("s")
        wid = s * 2 + c
        _zero_fill(z_v, s, acc, I1)
        ov = jnp.ones((16,), F32)

        @pl.loop(0, CH)
        def _(r):
            for jj in range(I1 // 16):
                ones_v[r, pl.ds(jj * 16, 16)] = ov

        pltpu.sync_copy(dstg.at[wid], dst_v)
        plsc.subcore_barrier()

        @pl.loop(0, NCH)
        def _(j):
            pltpu.sync_copy(ones_v, acc.at[dst_v.at[j]], add=True)

        plsc.subcore_barrier()
        pltpu.sync_copy(acc.at[pl.ds(s * RT, RT)],
                        out_hbm.at[c].at[pl.ds(s * RT, RT)])

    return pl.kernel(
        body,
        out_type=jax.ShapeDtypeStruct((2, NP, I1), F32),
        mesh=mesh,
        scratch_types=[
            pltpu.VMEM((NCH, CH), jnp.int32),
            pltpu.VMEM((CH, I1), F32),
            pltpu.VMEM((ZR, I1), F32),
            pltpu.VMEM_SHARED((NP, I1), F32),
        ],
    )


def kernel(x, params, edge_index, arch_list):
    p = params
    shift = ((arch_list[0] - A0) + (arch_list[1] - A1)).astype(F32)
    xp = jnp.pad(x + shift, ((0, NP - N), (0, 0)))
    ei = jnp.concatenate(
        [edge_index,
         jnp.full((2, EP - E), NP - 1, edge_index.dtype)], axis=1)
    srcg = ei[0].reshape(NW, NCH, CH)
    dstg = ei[1].reshape(NW, NCH, CH)

    degp = _sc_deg()(dstg)
    degs = degp[0, :, :1] + degp[1, :, :1]

    wcat1 = jnp.concatenate(
        [p['q1_w'].T, p['k1_w'].T, p['v1_w'].T, p['g1_w'].T], axis=1)
    bcat1 = jnp.concatenate(
        [p['q1_b'], p['k1_b'], p['v1_b'], jnp.zeros((I1,), F32)]
    ).reshape(1, 4 * I1)
    q1, k1, v1, hp1, dinv = _proj1(xp, wcat1, bcat1, degs)

    rel1 = _attn(q1, k1, v1, I1)
    s1p = _sc_scatter(I1)(hp1, srcg, dstg)
    s1a, s1b = s1p[0], s1p[1]

    wcat2 = jnp.concatenate(
        [p['q2_w'].T, p['k2_w'].T, p['v2_w'].T, p['g2_w'].T], axis=1)
    bcat2 = jnp.concatenate(
        [p['q2_b'], p['k2_b'], p['v2_b'], jnp.zeros((O2,), F32)]
    ).reshape(1, 4 * O2)
    q2, k2, v2, hp2 = _sq1(s1a, s1b, hp1, dinv, p['g1_b'].reshape(1, I1),
                           rel1, p['sq1_w'].T, p['sq1_b'].reshape(1, I1),
                           wcat2, bcat2)

    rel2 = _attn(q2, k2, v2, O2)
    s2p = _sc_scatter(I1)(hp2, srcg, dstg)
    s2a, s2b = s2p[0], s2p[1]

    xsq2 = _sq2(s2a, s2b, hp2, dinv, p['g2_b'].reshape(1, O2), rel2,
                p['sq2_w'].T, p['sq2_b'].reshape(1, O2))

    bottom = xsq2[1 + A0:1 + A0 + A1]
    br = jnp.pad(bottom.reshape(A0, 4 * O2), ((0, BR - A0), (0, 0)))
    wr = jnp.transpose(p['conv_w'], (2, 1, 0)).reshape(4 * O2, 4)
    cb = p['conv_b'].reshape(1, 4)

    (m2, tree, mid_avg, bwf, otw, omid, obot, feat, ogcn) = _tail(
        xsq2, br, wr, cb, p)

    tw = xsq2[0:1, :]
    mid = xsq2[1:1 + A0, :]
    bwv = m2.reshape(4 * BR, 1)[:A1]
    tree_bottom = tree[:A0]
    return (ogcn, otw, omid, obot, feat, tw, mid_avg, bwf, bwv, mid,
            tree_bottom)


# speedup vs baseline: 6.2247x; 1.0003x over previous
import functools

import jax
import jax.numpy as jnp
from jax import lax
from jax.experimental import pallas as pl
from jax.experimental.pallas import tpu as pltpu
from jax.experimental.pallas import tpu_sc as plsc

N = 10000
D = 256
E = 160000
I1 = 128
O2 = 64
A0 = 1999
A1 = 7996

NP = 10240
RB = 256
NB = NP // RB

F32 = jnp.float32


def _dot(a, b):
    return jnp.dot(a, b, preferred_element_type=F32)


def _dot_t(a, b):
    return jax.lax.dot_general(a, b, (((1,), (1,)), ((), ())),
                               preferred_element_type=F32)


def _proj1_body(x_ref, w_ref, b_ref, degs_ref, q_ref, k_ref, v_ref, hp_ref,
                dinv_ref):
    x = x_ref[...]
    y = _dot(x, w_ref[...]) + b_ref[...]
    deg = degs_ref[...] + 1.0
    dinv = jax.lax.rsqrt(deg)
    q_ref[...] = y[:, 0:I1]
    k_ref[...] = y[:, I1:2 * I1]
    v_ref[...] = y[:, 2 * I1:3 * I1]
    hp_ref[...] = dinv * y[:, 3 * I1:4 * I1]
    dinv_ref[...] = dinv


def _proj1(xp, wcat, bcat, degs):
    return pl.pallas_call(
        _proj1_body,
        grid=(NB,),
        in_specs=[
            pl.BlockSpec((RB, D), lambda i: (i, 0)),
            pl.BlockSpec((D, 4 * I1), lambda i: (0, 0)),
            pl.BlockSpec((1, 4 * I1), lambda i: (0, 0)),
            pl.BlockSpec((RB, 1), lambda i: (i, 0)),
        ],
        out_specs=[
            pl.BlockSpec((RB, I1), lambda i: (i, 0)),
            pl.BlockSpec((RB, I1), lambda i: (i, 0)),
            pl.BlockSpec((RB, I1), lambda i: (i, 0)),
            pl.BlockSpec((RB, I1), lambda i: (i, 0)),
            pl.BlockSpec((RB, 1), lambda i: (i, 0)),
        ],
        out_shape=[
            jax.ShapeDtypeStruct((NP, I1), F32),
            jax.ShapeDtypeStruct((NP, I1), F32),
            jax.ShapeDtypeStruct((NP, I1), F32),
            jax.ShapeDtypeStruct((NP, I1), F32),
            jax.ShapeDtypeStruct((NP, 1), F32),
        ],
    )(xp, wcat, bcat, degs)


def _attn_body(q_ref, k_ref, v_ref, o_ref):
    q = q_ref[...]
    s = _dot_t(q, k_ref[...])
    col = jax.lax.broadcasted_iota(jnp.int32, s.shape, 1)
    s = jnp.where(col < N, s, -3e38)
    m = jnp.max(s, axis=1, keepdims=True)
    p = jnp.exp(s - m)
    denom = jnp.sum(p, axis=1, keepdims=True)
    o = _dot(p, v_ref[...])
    o_ref[...] = o / denom


def _attn(q, k, v, dh):
    return pl.pallas_call(
        _attn_body,
        grid=(NB,),
        in_specs=[
            pl.BlockSpec((RB, dh), lambda i: (i, 0)),
            pl.BlockSpec((NP, dh), lambda i: (0, 0)),
            pl.BlockSpec((NP, dh), lambda i: (0, 0)),
        ],
        out_specs=pl.BlockSpec((RB, dh), lambda i: (i, 0)),
        out_shape=jax.ShapeDtypeStruct((NP, dh), F32),
    )(q, k, v)


def _sq1_body(s0_ref, s1_ref, hp_ref, dinv_ref, gb_ref, rel_ref, w1_ref,
              b1_ref, w2_ref, b2_ref, q_ref, k_ref, v_ref, hp2_ref):
    dinv = dinv_ref[...]
    xg = dinv * (s0_ref[...] + s1_ref[...] + hp_ref[...]) + gb_ref[...]
    a = jax.nn.relu(jnp.concatenate([xg, rel_ref[...]], axis=1))
    xsq = jax.nn.relu(_dot(a, w1_ref[...]) + b1_ref[...])
    y = _dot(xsq, w2_ref[...]) + b2_ref[...]
    q_ref[...] = y[:, 0:O2]
    k_ref[...] = y[:, O2:2 * O2]
    v_ref[...] = y[:, 2 * O2:3 * O2]
    hp2_ref[...] = jnp.concatenate(
        [dinv * y[:, 3 * O2:4 * O2], jnp.zeros((RB, O2), F32)], axis=1)


def _sq1(s0, s1, hp, dinv, gb, rel, w1t, b1, w2t, b2):
    return pl.pallas_call(
        _sq1_body,
        grid=(NB,),
        in_specs=[
            pl.BlockSpec((RB, I1), lambda i: (i, 0)),
            pl.BlockSpec((RB, I1), lambda i: (i, 0)),
            pl.BlockSpec((RB, I1), lambda i: (i, 0)),
            pl.BlockSpec((RB, 1), lambda i: (i, 0)),
            pl.BlockSpec((1, I1), lambda i: (0, 0)),
            pl.BlockSpec((RB, I1), lambda i: (i, 0)),
            pl.BlockSpec((2 * I1, I1), lambda i: (0, 0)),
            pl.BlockSpec((1, I1), lambda i: (0, 0)),
            pl.BlockSpec((I1, 4 * O2), lambda i: (0, 0)),
            pl.BlockSpec((1, 4 * O2), lambda i: (0, 0)),
        ],
        out_specs=[
            pl.BlockSpec((RB, O2), lambda i: (i, 0)),
            pl.BlockSpec((RB, O2), lambda i: (i, 0)),
            pl.BlockSpec((RB, O2), lambda i: (i, 0)),
            pl.BlockSpec((RB, I1), lambda i: (i, 0)),
        ],
        out_shape=[
            jax.ShapeDtypeStruct((NP, O2), F32),
            jax.ShapeDtypeStruct((NP, O2), F32),
            jax.ShapeDtypeStruct((NP, O2), F32),
            jax.ShapeDtypeStruct((NP, I1), F32),
        ],
    )(s0, s1, hp, dinv, gb, rel, w1t, b1, w2t, b2)


def _sq2_body(s0_ref, s1_ref, hp_ref, dinv_ref, gb_ref, rel_ref, w1_ref,
              b1_ref, o_ref):
    dinv = dinv_ref[...]
    acc = s0_ref[...] + s1_ref[...] + hp_ref[...]
    xg = dinv * acc[:, :O2] + gb_ref[...]
    a = jax.nn.relu(jnp.concatenate([xg, rel_ref[...]], axis=1))
    o_ref[...] = jax.nn.relu(_dot(a, w1_ref[...]) + b1_ref[...])


def _sq2(s0, s1, hp, dinv, gb, rel, w1t, b1):
    return pl.pallas_call(
        _sq2_body,
        grid=(NB,),
        in_specs=[
            pl.BlockSpec((RB, I1), lambda i: (i, 0)),
            pl.BlockSpec((RB, I1), lambda i: (i, 0)),
            pl.BlockSpec((RB, I1), lambda i: (i, 0)),
            pl.BlockSpec((RB, 1), lambda i: (i, 0)),
            pl.BlockSpec((1, O2), lambda i: (0, 0)),
            pl.BlockSpec((RB, O2), lambda i: (i, 0)),
            pl.BlockSpec((2 * O2, O2), lambda i: (0, 0)),
            pl.BlockSpec((1, O2), lambda i: (0, 0)),
        ],
        out_specs=pl.BlockSpec((RB, O2), lambda i: (i, 0)),
        out_shape=jax.ShapeDtypeStruct((NP, O2), F32),
    )(s0, s1, hp, dinv, gb, rel, w1t, b1)


BR = 2048


def _tail_body(xs_ref, br_ref, wr_ref, cb_ref, tww_ref, twb_ref, midw_ref,
               midb_ref, botw_ref, botb_ref, f1w_ref, f1b_ref, f2w_ref,
               f2b_ref, m2_ref, tree_ref, mavg_ref, bwf_ref, otw_ref,
               omid_ref, obot_ref, feat_ref, ogcn_ref):
    xs = xs_ref[...]
    tw = xs[0:1, :]
    row = jax.lax.broadcasted_iota(jnp.int32, (NP, 1), 0)
    wmid = jnp.where((row >= 1) & (row <= A0), 1.0 / A0, 0.0)
    mid_avg = jnp.sum(wmid * xs, axis=0, keepdims=True)

    b4 = br_ref[...]
    m = _dot(b4, wr_ref[...]) + cb_ref[...]
    grow = jax.lax.broadcasted_iota(jnp.int32, (BR, 1), 0)
    gmask = grow < A0
    mmax = jnp.max(jnp.where(gmask, m, -3e38))
    m1 = m / mmax
    rs = jnp.sum(jnp.abs(m1), axis=1, keepdims=True)
    m2 = jnp.where(gmask, m1 / rs, 0.0)
    tree = (m2[:, 0:1] * b4[:, 0:O2]
            + m2[:, 1:2] * b4[:, O2:2 * O2]
            + m2[:, 2:3] * b4[:, 2 * O2:3 * O2]
            + m2[:, 3:4] * b4[:, 3 * O2:4 * O2])
    bwf = jnp.sum(tree, axis=0, keepdims=True) * (1.0 / A0)

    otw = _dot(tw, tww_ref[...]) + twb_ref[...]
    omid = _dot(mid_avg, midw_ref[...]) + midb_ref[...]
    obot = _dot(bwf, botw_ref[...]) + botb_ref[...]
    feat = _dot(jnp.concatenate([tw, mid_avg, bwf], axis=1),
                f1w_ref[...]) + f1b_ref[...]
    ogcn = _dot(jax.nn.relu(feat), f2w_ref[...]) + f2b_ref[...]

    m2_ref[...] = m2
    tree_ref[...] = tree
    mavg_ref[...] = mid_avg
    bwf_ref[...] = bwf
    otw_ref[...] = otw
    omid_ref[...] = omid
    obot_ref[...] = obot
    feat_ref[...] = feat
    ogcn_ref[...] = ogcn


def _tail(xs, br, wr, cb, p):
    full = lambda shape: pl.BlockSpec(shape, lambda: tuple(0 for _ in shape))
    ins = [
        (xs, (NP, O2)), (br, (BR, 4 * O2)), (wr, (4 * O2, 4)), (cb, (1, 4)),
        (p['tw_w'].T, (O2, 4)), (p['tw_b'].reshape(1, 4), (1, 4)),
        (p['mid_w'].T, (O2, 4)), (p['mid_b'].reshape(1, 4), (1, 4)),
        (p['bot_w'].T, (O2, 4)), (p['bot_b'].reshape(1, 4), (1, 4)),
        (p['fc1_w'].T, (3 * O2, O2)), (p['fc1_b'].reshape(1, O2), (1, O2)),
        (p['fc2_w'].T, (O2, 4)), (p['fc2_b'].reshape(1, 4), (1, 4)),
    ]
    outs = [
        ((BR, 4), F32), ((BR, O2), F32), ((1, O2), F32), ((1, O2), F32),
        ((1, 4), F32), ((1, 4), F32), ((1, 4), F32), ((1, O2), F32),
        ((1, 4), F32),
    ]
    return pl.pallas_call(
        _tail_body,
        in_specs=[full(s) for _, s in ins],
        out_specs=[full(s) for s, _ in outs],
        out_shape=[jax.ShapeDtypeStruct(s, d) for s, d in outs],
    )(*[a for a, _ in ins])


NW = 32
CH = 128
NCH = 40
EP = NW * NCH * CH
RT = NP // 16
ZR = 32


def _zero_fill(z_v, s, acc, dm):
    zv = jnp.zeros((16,), F32)

    @pl.loop(0, ZR)
    def _(r):
        for jj in range(dm // 16):
            z_v[r, pl.ds(jj * 16, 16)] = zv

    @pl.loop(0, RT // ZR)
    def _(t):
        pltpu.sync_copy(z_v, acc.at[pl.ds(s * RT + t * ZR, ZR)])


def _sc_scatter(dm):
    mesh = plsc.VectorSubcoreMesh(core_axis_name="c", subcore_axis_name="s")

    def body(hp_hbm, srcg, dstg, out_hbm, src_v, dst_v, rows0, rows1, z_v,
             acc, sem0, sem1):
        c = lax.axis_index("c")
        s = lax.axis_index("s")
        wid = s * 2 + c
        _zero_fill(z_v, s, acc, dm)
        pltpu.sync_copy(srcg.at[wid], src_v)
        pltpu.sync_copy(dstg.at[wid], dst_v)
        plsc.subcore_barrier()

        def gather(j, buf, sem):
            pltpu.async_copy(hp_hbm.at[src_v.at[j]], buf, sem)

        def drain(buf, sem):
            pltpu.make_async_copy(hp_hbm.at[src_v.at[0]], buf, sem).wait()

        def scat(j, buf):
            pltpu.sync_copy(buf, acc.at[dst_v.at[j]], add=True)

        jmax = jnp.minimum(NCH, E // CH - wid * NCH)
        gather(0, rows0, sem0)

        @pl.loop(0, jmax, step=2)
        def _(j):
            gather(j + 1, rows1, sem1)
            drain(rows0, sem0)
            scat(j, rows0)

            @pl.when(j + 2 < jmax)
            def _():
                gather(j + 2, rows0, sem0)

            drain(rows1, sem1)
            scat(j + 1, rows1)

        plsc.subcore_barrier()
        pltpu.sync_copy(acc.at[pl.ds(s * RT, RT)],
                        out_hbm.at[c].at[pl.ds(s * RT, RT)])

    return pl.kernel(
        body,
        out_type=jax.ShapeDtypeStruct((2, NP, dm), F32),
        mesh=mesh,
        scratch_types=[
            pltpu.VMEM((NCH, CH), jnp.int32),
            pltpu.VMEM((NCH, CH), jnp.int32),
            pltpu.VMEM((CH, dm), F32),
            pltpu.VMEM((CH, dm), F32),
            pltpu.VMEM((ZR, dm), F32),
            pltpu.VMEM_SHARED((NP, dm), F32),
            pltpu.SemaphoreType.DMA,
            pltpu.SemaphoreType.DMA,
        ],
    )


def _sc_deg():
    mesh = plsc.VectorSubcoreMesh(core_axis_name="c", subcore_axis_name="s")

    def body(dstg, out_hbm, dst_v, ones_v, z_v, acc):
        c = lax.axis_index("c")
        s = lax.axis_index("s")
        wid = s * 2 + c
        _zero_fill(z_v, s, acc, I1)
        ov = jnp.ones((16,), F32)

        @pl.loop(0, CH)
        def _(r):
            for jj in range(I1 // 16):
                ones_v[r, pl.ds(jj * 16, 16)] = ov

        pltpu.sync_copy(dstg.at[wid], dst_v)
        plsc.subcore_barrier()
        jmax = jnp.minimum(NCH, E // CH - wid * NCH)

        @pl.loop(0, jmax)
        def _(j):
            pltpu.sync_copy(ones_v, acc.at[dst_v.at[j]], add=True)

        plsc.subcore_barrier()
        pltpu.sync_copy(acc.at[pl.ds(s * RT, RT)],
                        out_hbm.at[c].at[pl.ds(s * RT, RT)])

    return pl.kernel(
        body,
        out_type=jax.ShapeDtypeStruct((2, NP, I1), F32),
        mesh=mesh,
        scratch_types=[
            pltpu.VMEM((NCH, CH), jnp.int32),
            pltpu.VMEM((CH, I1), F32),
            pltpu.VMEM((ZR, I1), F32),
            pltpu.VMEM_SHARED((NP, I1), F32),
        ],
    )


def kernel(x, params, edge_index, arch_list):
    p = params
    shift = ((arch_list[0] - A0) + (arch_list[1] - A1)).astype(F32)
    xp = jnp.pad(x + shift, ((0, NP - N), (0, 0)))
    ei = jnp.concatenate(
        [edge_index,
         jnp.full((2, EP - E), NP - 1, edge_index.dtype)], axis=1)
    srcg = ei[0].reshape(NW, NCH, CH)
    dstg = ei[1].reshape(NW, NCH, CH)

    degp = _sc_deg()(dstg)
    degs = degp[0, :, :1] + degp[1, :, :1]

    wcat1 = jnp.concatenate(
        [p['q1_w'].T, p['k1_w'].T, p['v1_w'].T, p['g1_w'].T], axis=1)
    bcat1 = jnp.concatenate(
        [p['q1_b'], p['k1_b'], p['v1_b'], jnp.zeros((I1,), F32)]
    ).reshape(1, 4 * I1)
    q1, k1, v1, hp1, dinv = _proj1(xp, wcat1, bcat1, degs)

    rel1 = _attn(q1, k1, v1, I1)
    s1p = _sc_scatter(I1)(hp1, srcg, dstg)
    s1a, s1b = s1p[0], s1p[1]

    wcat2 = jnp.concatenate(
        [p['q2_w'].T, p['k2_w'].T, p['v2_w'].T, p['g2_w'].T], axis=1)
    bcat2 = jnp.concatenate(
        [p['q2_b'], p['k2_b'], p['v2_b'], jnp.zeros((O2,), F32)]
    ).reshape(1, 4 * O2)
    q2, k2, v2, hp2 = _sq1(s1a, s1b, hp1, dinv, p['g1_b'].reshape(1, I1),
                           rel1, p['sq1_w'].T, p['sq1_b'].reshape(1, I1),
                           wcat2, bcat2)

    rel2 = _attn(q2, k2, v2, O2)
    s2p = _sc_scatter(I1)(hp2, srcg, dstg)
    s2a, s2b = s2p[0], s2p[1]

    xsq2 = _sq2(s2a, s2b, hp2, dinv, p['g2_b'].reshape(1, O2), rel2,
                p['sq2_w'].T, p['sq2_b'].reshape(1, O2))

    bottom = xsq2[1 + A0:1 + A0 + A1]
    br = jnp.pad(bottom.reshape(A0, 4 * O2), ((0, BR - A0), (0, 0)))
    wr = jnp.transpose(p['conv_w'], (2, 1, 0)).reshape(4 * O2, 4)
    cb = p['conv_b'].reshape(1, 4)

    (m2, tree, mid_avg, bwf, otw, omid, obot, feat, ogcn) = _tail(
        xsq2, br, wr, cb, p)

    tw = xsq2[0:1, :]
    mid = xsq2[1:1 + A0, :]
    bwv = m2.reshape(4 * BR, 1)[:A1]
    tree_bottom = tree[:A0]
    return (ogcn, otw, omid, obot, feat, tw, mid_avg, bwf, bwv, mid,
            tree_bottom)


# speedup vs baseline: 8.2007x; 1.3174x over previous
import functools

import jax
import jax.numpy as jnp
from jax import lax
from jax.experimental import pallas as pl
from jax.experimental.pallas import tpu as pltpu
from jax.experimental.pallas import tpu_sc as plsc

N = 10000
D = 256
E = 160000
I1 = 128
O2 = 64
A0 = 1999
A1 = 7996

NP = 10240
RB = 256
NB = NP // RB

F32 = jnp.float32


def _dot(a, b):
    return jnp.dot(a, b, preferred_element_type=F32)


def _dot_t(a, b):
    return jax.lax.dot_general(a, b, (((1,), (1,)), ((), ())),
                               preferred_element_type=F32)


def _proj1_body(x_ref, w_ref, b_ref, degs_ref, q_ref, k_ref, v_ref, hp_ref,
                dinv_ref):
    x = x_ref[...]
    y = _dot(x, w_ref[...]) + b_ref[...]
    deg = degs_ref[...] + 1.0
    dinv = jax.lax.rsqrt(deg)
    q_ref[...] = y[:, 0:I1]
    k_ref[...] = y[:, I1:2 * I1]
    v_ref[...] = y[:, 2 * I1:3 * I1]
    hp_ref[...] = dinv * y[:, 3 * I1:4 * I1]
    dinv_ref[...] = dinv


def _proj1(xp, wcat, bcat, degs):
    return pl.pallas_call(
        _proj1_body,
        grid=(NB,),
        in_specs=[
            pl.BlockSpec((RB, D), lambda i: (i, 0)),
            pl.BlockSpec((D, 4 * I1), lambda i: (0, 0)),
            pl.BlockSpec((1, 4 * I1), lambda i: (0, 0)),
            pl.BlockSpec((RB, 1), lambda i: (i, 0)),
        ],
        out_specs=[
            pl.BlockSpec((RB, I1), lambda i: (i, 0)),
            pl.BlockSpec((RB, I1), lambda i: (i, 0)),
            pl.BlockSpec((RB, I1), lambda i: (i, 0)),
            pl.BlockSpec((RB, I1), lambda i: (i, 0)),
            pl.BlockSpec((RB, 1), lambda i: (i, 0)),
        ],
        out_shape=[
            jax.ShapeDtypeStruct((NP, I1), F32),
            jax.ShapeDtypeStruct((NP, I1), F32),
            jax.ShapeDtypeStruct((NP, I1), F32),
            jax.ShapeDtypeStruct((NP, I1), F32),
            jax.ShapeDtypeStruct((NP, 1), F32),
        ],
    )(xp, wcat, bcat, degs)


def _attn_body(q_ref, k_ref, v_ref, o_ref):
    q = q_ref[...]
    s = _dot_t(q, k_ref[...])
    col = jax.lax.broadcasted_iota(jnp.int32, s.shape, 1)
    s = jnp.where(col < N, s, -3e38)
    m = jnp.max(s, axis=1, keepdims=True)
    p = jnp.exp(s - m)
    denom = jnp.sum(p, axis=1, keepdims=True)
    o = _dot(p.astype(jnp.bfloat16), v_ref[...].astype(jnp.bfloat16))
    o_ref[...] = o / denom


def _attn(q, k, v, dh):
    return pl.pallas_call(
        _attn_body,
        grid=(NB,),
        in_specs=[
            pl.BlockSpec((RB, dh), lambda i: (i, 0)),
            pl.BlockSpec((NP, dh), lambda i: (0, 0)),
            pl.BlockSpec((NP, dh), lambda i: (0, 0)),
        ],
        out_specs=pl.BlockSpec((RB, dh), lambda i: (i, 0)),
        out_shape=jax.ShapeDtypeStruct((NP, dh), F32),
    )(q, k, v)


def _sq1_body(s0_ref, s1_ref, hp_ref, dinv_ref, gb_ref, rel_ref, w1_ref,
              b1_ref, w2_ref, b2_ref, q_ref, k_ref, v_ref, hp2_ref):
    dinv = dinv_ref[...]
    xg = dinv * (s0_ref[...] + s1_ref[...] + hp_ref[...]) + gb_ref[...]
    a = jax.nn.relu(jnp.concatenate([xg, rel_ref[...]], axis=1))
    xsq = jax.nn.relu(_dot(a, w1_ref[...]) + b1_ref[...])
    y = _dot(xsq, w2_ref[...]) + b2_ref[...]
    q_ref[...] = y[:, 0:O2]
    k_ref[...] = y[:, O2:2 * O2]
    v_ref[...] = y[:, 2 * O2:3 * O2]
    hp2_ref[...] = jnp.concatenate(
        [dinv * y[:, 3 * O2:4 * O2], jnp.zeros((RB, O2), F32)], axis=1)


def _sq1(s0, s1, hp, dinv, gb, rel, w1t, b1, w2t, b2):
    return pl.pallas_call(
        _sq1_body,
        grid=(NB,),
        in_specs=[
            pl.BlockSpec((RB, I1), lambda i: (i, 0)),
            pl.BlockSpec((RB, I1), lambda i: (i, 0)),
            pl.BlockSpec((RB, I1), lambda i: (i, 0)),
            pl.BlockSpec((RB, 1), lambda i: (i, 0)),
            pl.BlockSpec((1, I1), lambda i: (0, 0)),
            pl.BlockSpec((RB, I1), lambda i: (i, 0)),
            pl.BlockSpec((2 * I1, I1), lambda i: (0, 0)),
            pl.BlockSpec((1, I1), lambda i: (0, 0)),
            pl.BlockSpec((I1, 4 * O2), lambda i: (0, 0)),
            pl.BlockSpec((1, 4 * O2), lambda i: (0, 0)),
        ],
        out_specs=[
            pl.BlockSpec((RB, O2), lambda i: (i, 0)),
            pl.BlockSpec((RB, O2), lambda i: (i, 0)),
            pl.BlockSpec((RB, O2), lambda i: (i, 0)),
            pl.BlockSpec((RB, I1), lambda i: (i, 0)),
        ],
        out_shape=[
            jax.ShapeDtypeStruct((NP, O2), F32),
            jax.ShapeDtypeStruct((NP, O2), F32),
            jax.ShapeDtypeStruct((NP, O2), F32),
            jax.ShapeDtypeStruct((NP, I1), F32),
        ],
    )(s0, s1, hp, dinv, gb, rel, w1t, b1, w2t, b2)


def _sq2_body(s0_ref, s1_ref, hp_ref, dinv_ref, gb_ref, rel_ref, w1_ref,
              b1_ref, o_ref):
    dinv = dinv_ref[...]
    acc = s0_ref[...] + s1_ref[...] + hp_ref[...]
    xg = dinv * acc[:, :O2] + gb_ref[...]
    a = jax.nn.relu(jnp.concatenate([xg, rel_ref[...]], axis=1))
    o_ref[...] = jax.nn.relu(_dot(a, w1_ref[...]) + b1_ref[...])


def _sq2(s0, s1, hp, dinv, gb, rel, w1t, b1):
    return pl.pallas_call(
        _sq2_body,
        grid=(NB,),
        in_specs=[
            pl.BlockSpec((RB, I1), lambda i: (i, 0)),
            pl.BlockSpec((RB, I1), lambda i: (i, 0)),
            pl.BlockSpec((RB, I1), lambda i: (i, 0)),
            pl.BlockSpec((RB, 1), lambda i: (i, 0)),
            pl.BlockSpec((1, O2), lambda i: (0, 0)),
            pl.BlockSpec((RB, O2), lambda i: (i, 0)),
            pl.BlockSpec((2 * O2, O2), lambda i: (0, 0)),
            pl.BlockSpec((1, O2), lambda i: (0, 0)),
        ],
        out_specs=pl.BlockSpec((RB, O2), lambda i: (i, 0)),
        out_shape=jax.ShapeDtypeStruct((NP, O2), F32),
    )(s0, s1, hp, dinv, gb, rel, w1t, b1)


BR = 2048


def _tail_body(xs_ref, br_ref, wr_ref, cb_ref, tww_ref, twb_ref, midw_ref,
               midb_ref, botw_ref, botb_ref, f1w_ref, f1b_ref, f2w_ref,
               f2b_ref, m2_ref, tree_ref, mavg_ref, bwf_ref, otw_ref,
               omid_ref, obot_ref, feat_ref, ogcn_ref):
    xs = xs_ref[...]
    tw = xs[0:1, :]
    row = jax.lax.broadcasted_iota(jnp.int32, (NP, 1), 0)
    wmid = jnp.where((row >= 1) & (row <= A0), 1.0 / A0, 0.0)
    mid_avg = jnp.sum(wmid * xs, axis=0, keepdims=True)

    b4 = br_ref[...]
    m = _dot(b4, wr_ref[...]) + cb_ref[...]
    grow = jax.lax.broadcasted_iota(jnp.int32, (BR, 1), 0)
    gmask = grow < A0
    mmax = jnp.max(jnp.where(gmask, m, -3e38))
    m1 = m / mmax
    rs = jnp.sum(jnp.abs(m1), axis=1, keepdims=True)
    m2 = jnp.where(gmask, m1 / rs, 0.0)
    tree = (m2[:, 0:1] * b4[:, 0:O2]
            + m2[:, 1:2] * b4[:, O2:2 * O2]
            + m2[:, 2:3] * b4[:, 2 * O2:3 * O2]
            + m2[:, 3:4] * b4[:, 3 * O2:4 * O2])
    bwf = jnp.sum(tree, axis=0, keepdims=True) * (1.0 / A0)

    otw = _dot(tw, tww_ref[...]) + twb_ref[...]
    omid = _dot(mid_avg, midw_ref[...]) + midb_ref[...]
    obot = _dot(bwf, botw_ref[...]) + botb_ref[...]
    feat = _dot(jnp.concatenate([tw, mid_avg, bwf], axis=1),
                f1w_ref[...]) + f1b_ref[...]
    ogcn = _dot(jax.nn.relu(feat), f2w_ref[...]) + f2b_ref[...]

    m2_ref[...] = m2
    tree_ref[...] = tree
    mavg_ref[...] = mid_avg
    bwf_ref[...] = bwf
    otw_ref[...] = otw
    omid_ref[...] = omid
    obot_ref[...] = obot
    feat_ref[...] = feat
    ogcn_ref[...] = ogcn


def _tail(xs, br, wr, cb, p):
    full = lambda shape: pl.BlockSpec(shape, lambda: tuple(0 for _ in shape))
    ins = [
        (xs, (NP, O2)), (br, (BR, 4 * O2)), (wr, (4 * O2, 4)), (cb, (1, 4)),
        (p['tw_w'].T, (O2, 4)), (p['tw_b'].reshape(1, 4), (1, 4)),
        (p['mid_w'].T, (O2, 4)), (p['mid_b'].reshape(1, 4), (1, 4)),
        (p['bot_w'].T, (O2, 4)), (p['bot_b'].reshape(1, 4), (1, 4)),
        (p['fc1_w'].T, (3 * O2, O2)), (p['fc1_b'].reshape(1, O2), (1, O2)),
        (p['fc2_w'].T, (O2, 4)), (p['fc2_b'].reshape(1, 4), (1, 4)),
    ]
    outs = [
        ((BR, 4), F32), ((BR, O2), F32), ((1, O2), F32), ((1, O2), F32),
        ((1, 4), F32), ((1, 4), F32), ((1, 4), F32), ((1, O2), F32),
        ((1, 4), F32),
    ]
    return pl.pallas_call(
        _tail_body,
        in_specs=[full(s) for _, s in ins],
        out_specs=[full(s) for s, _ in outs],
        out_shape=[jax.ShapeDtypeStruct(s, d) for s, d in outs],
    )(*[a for a, _ in ins])


NW = 32
CH = 128
NCH = 40
EP = NW * NCH * CH
RT = NP // 16
ZR = 32


def _zero_fill(z_v, s, acc, dm):
    zv = jnp.zeros((16,), F32)

    @pl.loop(0, ZR)
    def _(r):
        for jj in range(dm // 16):
            z_v[r, pl.ds(jj * 16, 16)] = zv

    @pl.loop(0, RT // ZR)
    def _(t):
        pltpu.sync_copy(z_v, acc.at[pl.ds(s * RT + t * ZR, ZR)])


def _sc_scatter(dm):
    mesh = plsc.VectorSubcoreMesh(core_axis_name="c", subcore_axis_name="s")

    def body(hp_hbm, srcg, dstg, out_hbm, src_v, dst_v, rows0, rows1, z_v,
             acc, sem0, sem1):
        c = lax.axis_index("c")
        s = lax.axis_index("s")
        wid = s * 2 + c
        _zero_fill(z_v, s, acc, dm)
        pltpu.sync_copy(srcg.at[wid], src_v)
        pltpu.sync_copy(dstg.at[wid], dst_v)
        plsc.subcore_barrier()

        def gather(j, buf, sem):
            pltpu.async_copy(hp_hbm.at[src_v.at[j]], buf, sem)

        def drain(buf, sem):
            pltpu.make_async_copy(hp_hbm.at[src_v.at[0]], buf, sem).wait()

        def scat(j, buf):
            pltpu.sync_copy(buf, acc.at[dst_v.at[j]], add=True)

        jmax = jnp.minimum(NCH, E // CH - wid * NCH)
        gather(0, rows0, sem0)

        @pl.loop(0, jmax, step=2)
        def _(j):
            gather(j + 1, rows1, sem1)
            drain(rows0, sem0)
            scat(j, rows0)

            @pl.when(j + 2 < jmax)
            def _():
                gather(j + 2, rows0, sem0)

            drain(rows1, sem1)
            scat(j + 1, rows1)

        plsc.subcore_barrier()
        pltpu.sync_copy(acc.at[pl.ds(s * RT, RT)],
                        out_hbm.at[c].at[pl.ds(s * RT, RT)])

    return pl.kernel(
        body,
        out_type=jax.ShapeDtypeStruct((2, NP, dm), F32),
        mesh=mesh,
        scratch_types=[
            pltpu.VMEM((NCH, CH), jnp.int32),
            pltpu.VMEM((NCH, CH), jnp.int32),
            pltpu.VMEM((CH, dm), F32),
            pltpu.VMEM((CH, dm), F32),
            pltpu.VMEM((ZR, dm), F32),
            pltpu.VMEM_SHARED((NP, dm), F32),
            pltpu.SemaphoreType.DMA,
            pltpu.SemaphoreType.DMA,
        ],
    )


def _sc_deg():
    mesh = plsc.VectorSubcoreMesh(core_axis_name="c", subcore_axis_name="s")

    def body(dstg, out_hbm, dst_v, ones_v, z_v, acc):
        c = lax.axis_index("c")
        s = lax.axis_index("s")
        wid = s * 2 + c
        _zero_fill(z_v, s, acc, I1)
        ov = jnp.ones((16,), F32)

        @pl.loop(0, CH)
        def _(r):
            for jj in range(I1 // 16):
                ones_v[r, pl.ds(jj * 16, 16)] = ov

        pltpu.sync_copy(dstg.at[wid], dst_v)
        plsc.subcore_barrier()
        jmax = jnp.minimum(NCH, E // CH - wid * NCH)

        @pl.loop(0, jmax)
        def _(j):
            pltpu.sync_copy(ones_v, acc.at[dst_v.at[j]], add=True)

        plsc.subcore_barrier()
        pltpu.sync_copy(acc.at[pl.ds(s * RT, RT)],
                        out_hbm.at[c].at[pl.ds(s * RT, RT)])

    return pl.kernel(
        body,
        out_type=jax.ShapeDtypeStruct((2, NP, I1), F32),
        mesh=mesh,
        scratch_types=[
            pltpu.VMEM((NCH, CH), jnp.int32),
            pltpu.VMEM((CH, I1), F32),
            pltpu.VMEM((ZR, I1), F32),
            pltpu.VMEM_SHARED((NP, I1), F32),
        ],
    )


def kernel(x, params, edge_index, arch_list):
    p = params
    shift = ((arch_list[0] - A0) + (arch_list[1] - A1)).astype(F32)
    xp = jnp.pad(x + shift, ((0, NP - N), (0, 0)))
    ei = jnp.concatenate(
        [edge_index,
         jnp.full((2, EP - E), NP - 1, edge_index.dtype)], axis=1)
    srcg = ei[0].reshape(NW, NCH, CH)
    dstg = ei[1].reshape(NW, NCH, CH)

    degp = _sc_deg()(dstg)
    degs = degp[0, :, :1] + degp[1, :, :1]

    wcat1 = jnp.concatenate(
        [p['q1_w'].T, p['k1_w'].T, p['v1_w'].T, p['g1_w'].T], axis=1)
    bcat1 = jnp.concatenate(
        [p['q1_b'], p['k1_b'], p['v1_b'], jnp.zeros((I1,), F32)]
    ).reshape(1, 4 * I1)
    q1, k1, v1, hp1, dinv = _proj1(xp, wcat1, bcat1, degs)

    rel1 = _attn(q1, k1, v1, I1)
    s1p = _sc_scatter(I1)(hp1, srcg, dstg)
    s1a, s1b = s1p[0], s1p[1]

    wcat2 = jnp.concatenate(
        [p['q2_w'].T, p['k2_w'].T, p['v2_w'].T, p['g2_w'].T], axis=1)
    bcat2 = jnp.concatenate(
        [p['q2_b'], p['k2_b'], p['v2_b'], jnp.zeros((O2,), F32)]
    ).reshape(1, 4 * O2)
    q2, k2, v2, hp2 = _sq1(s1a, s1b, hp1, dinv, p['g1_b'].reshape(1, I1),
                           rel1, p['sq1_w'].T, p['sq1_b'].reshape(1, I1),
                           wcat2, bcat2)

    rel2 = _attn(q2, k2, v2, O2)
    s2p = _sc_scatter(I1)(hp2, srcg, dstg)
    s2a, s2b = s2p[0], s2p[1]

    xsq2 = _sq2(s2a, s2b, hp2, dinv, p['g2_b'].reshape(1, O2), rel2,
                p['sq2_w'].T, p['sq2_b'].reshape(1, O2))

    bottom = xsq2[1 + A0:1 + A0 + A1]
    br = jnp.pad(bottom.reshape(A0, 4 * O2), ((0, BR - A0), (0, 0)))
    wr = jnp.transpose(p['conv_w'], (2, 1, 0)).reshape(4 * O2, 4)
    cb = p['conv_b'].reshape(1, 4)

    (m2, tree, mid_avg, bwf, otw, omid, obot, feat, ogcn) = _tail(
        xsq2, br, wr, cb, p)

    tw = xsq2[0:1, :]
    mid = xsq2[1:1 + A0, :]
    bwv = m2.reshape(4 * BR, 1)[:A1]
    tree_bottom = tree[:A0]
    return (ogcn, otw, omid, obot, feat, tw, mid_avg, bwf, bwv, mid,
            tree_bottom)
